# Initial kernel scaffold; baseline (speedup 1.0000x reference)
import jax
import jax.numpy as jnp
from jax.experimental import pallas as pl


def kernel(co_e, ex_e, score, time, h0, vs, hs, W_resize, b_resize, Wk, bk, know_mem, Ws, bs, W_ih, W_hh, b_ih, b_hh):
    raise NotImplementedError("write your pallas kernel here")


# speedup vs baseline: 1.8083x; 1.8083x over previous
import jax
import jax.numpy as jnp
from jax.experimental import pallas as pl
from jax.experimental.pallas import tpu as pltpu

_T = 2048
_KL = 128
_H = 256
_KE = 64
_TS = 100
_EX = 768
_K = 64
_NEG = float("-inf")


def _small_kernel(ex_e_ref, co_e_ref, score_ref, h0_ref, vsT_ref,
                  WrT_ref, br_ref, WkT_ref, bk_ref, kmT_ref,
                  WihTv_ref, WihTs_ref, bih_ref, WhhT_ref, bhh_ref,
                  topic_ref, bsm_ref, idx_ref, alpha_ref, h_ref):
    topic = ex_e_ref[...] @ WrT_ref[...] + br_ref[...]
    topic_ref[...] = topic
    beta = topic @ vsT_ref[...]

    iota_t = jax.lax.broadcasted_iota(jnp.int32, (1, _T), 1)
    iota_k = jax.lax.broadcasted_iota(jnp.int32, (1, _K), 1)

    def body(k, carry):
        b, vals, idxs = carry
        m = jnp.max(b)
        im = jnp.min(jnp.where(b == m, iota_t, _T))
        b = jnp.where(iota_t == im, _NEG, b)
        sel = iota_k == k
        vals = jnp.where(sel, m, vals)
        idxs = jnp.where(sel, im, idxs)
        return b, vals, idxs

    vals0 = jnp.full((1, _K), _NEG, jnp.float32)
    idxs0 = jnp.zeros((1, _K), jnp.int32)
    _, vals, idxs = jax.lax.fori_loop(0, _K, body, (beta, vals0, idxs0))

    e = jnp.exp(vals - jnp.max(vals))
    bsm_ref[...] = e / jnp.sum(e)
    idx_ref[...] = idxs

    kn = co_e_ref[...] @ WkT_ref[...] + bk_ref[...]
    al = kn @ kmT_ref[...]
    ea = jnp.exp(al - jnp.max(al))
    alpha = ea / jnp.sum(ea)
    alpha_ref[...] = alpha

    g_row = topic @ WihTv_ref[...] + score_ref[0, 0] * WihTs_ref[...]
    alpha_col = alpha.reshape(_KL, 1)
    gi = alpha_col * g_row + bih_ref[...]
    hprev = h0_ref[...]
    gh = hprev @ WhhT_ref[...] + bhh_ref[...]
    r = jax.nn.sigmoid(gi[:, :_H] + gh[:, :_H])
    z = jax.nn.sigmoid(gi[:, _H:2 * _H] + gh[:, _H:2 * _H])
    n = jnp.tanh(gi[:, 2 * _H:] + r * gh[:, 2 * _H:])
    h_ref[...] = (1.0 - z) * n + z * hprev


def _gather_kernel(idx_ref, hs_ref, bsm_ref, alpha_ref, topic_ref,
                   Wsv_ref, Wsh_ref, bs_ref, out_ref, acc_ref):
    k = pl.program_id(0)
    iota_k = jax.lax.broadcasted_iota(jnp.int32, (1, _K), 1)
    w = jnp.sum(jnp.where(iota_k == k, bsm_ref[...], 0.0))
    blk = hs_ref[0]

    @pl.when(k == 0)
    def _():
        acc_ref[...] = w * blk

    @pl.when(k > 0)
    def _():
        acc_ref[...] = acc_ref[...] + w * blk

    @pl.when(k == _K - 1)
    def _():
        hkp = alpha_ref[...] @ acc_ref[...]
        pred = (jnp.sum(topic_ref[...] * Wsv_ref[...], axis=1, keepdims=True)
                + jnp.sum(hkp * Wsh_ref[...], axis=1, keepdims=True)
                + bs_ref[...])
        out_ref[...] = pred


def kernel(co_e, ex_e, score, time, h0, vs, hs, W_resize, b_resize, Wk, bk,
           know_mem, Ws, bs, W_ih, W_hh, b_ih, b_hh):
    co_e2 = co_e.reshape(1, _KL)
    score2 = score.reshape(1, 1)
    h02 = h0.reshape(_KL, _H)
    vsT = vs.T
    WrT = W_resize.T
    br2 = b_resize.reshape(1, _TS)
    WkT = Wk.T
    bk2 = bk.reshape(1, _KE)
    kmT = know_mem.T
    WihT = W_ih.T
    WihTv = WihT[:_TS]
    WihTs = WihT[_TS:]
    bih2 = b_ih.reshape(1, 3 * _H)
    WhhT = W_hh.T
    bhh2 = b_hh.reshape(1, 3 * _H)

    topic, bsm, idx, alpha, hnew = pl.pallas_call(
        _small_kernel,
        out_shape=(
            jax.ShapeDtypeStruct((1, _TS), jnp.float32),
            jax.ShapeDtypeStruct((1, _K), jnp.float32),
            jax.ShapeDtypeStruct((1, _K), jnp.int32),
            jax.ShapeDtypeStruct((1, _KL), jnp.float32),
            jax.ShapeDtypeStruct((_KL, _H), jnp.float32),
        ),
    )(ex_e, co_e2, score2, h02, vsT, WrT, br2, WkT, bk2, kmT,
      WihTv, WihTs, bih2, WhhT, bhh2)

    Wsv = Ws[:, :_TS]
    Wsh = Ws[:, _TS:]
    bs2 = bs.reshape(1, 1)

    grid_spec = pltpu.PrefetchScalarGridSpec(
        num_scalar_prefetch=1,
        grid=(_K,),
        in_specs=[
            pl.BlockSpec((1, _KL, _H), lambda k, idx_ref: (idx_ref[k], 0, 0)),
            pl.BlockSpec((1, _K), lambda k, idx_ref: (0, 0)),
            pl.BlockSpec((1, _KL), lambda k, idx_ref: (0, 0)),
            pl.BlockSpec((1, _TS), lambda k, idx_ref: (0, 0)),
            pl.BlockSpec((1, _TS), lambda k, idx_ref: (0, 0)),
            pl.BlockSpec((1, _H), lambda k, idx_ref: (0, 0)),
            pl.BlockSpec((1, 1), lambda k, idx_ref: (0, 0)),
        ],
        out_specs=pl.BlockSpec((1, 1), lambda k, idx_ref: (0, 0)),
        scratch_shapes=[pltpu.VMEM((_KL, _H), jnp.float32)],
    )
    pred = pl.pallas_call(
        _gather_kernel,
        grid_spec=grid_spec,
        out_shape=jax.ShapeDtypeStruct((1, 1), jnp.float32),
    )(idx.reshape(_K), hs, bsm, alpha, topic, Wsv, Wsh, bs2)

    h = hnew.reshape(1, _KL, _H)
    vs_new = jnp.concatenate([vs, topic], axis=0)
    hs_new = jnp.concatenate([hs, h], axis=0)
    return (pred.reshape(1), h, vs_new, hs_new, bsm)


# speedup vs baseline: 2.1250x; 1.1751x over previous
import jax
import jax.numpy as jnp
from jax.experimental import pallas as pl
from jax.experimental.pallas import tpu as pltpu

_T = 2048
_KL = 128
_H = 256
_KE = 64
_TS = 100
_EX = 768
_K = 64
_BT = 64
_NB = _T // _BT
_NEG = float("-inf")


def _fused_kernel(ex_e_ref, co_e_ref, score_ref, h0_ref, vsT_ref,
                  WrT_ref, br_ref, WkT_ref, bk_ref, kmT_ref,
                  WihTv_ref, WihTs_ref, bih_ref, WhhT_ref, bhh_ref,
                  Wsv_ref, Wsh_ref, bs_ref, hs_ref,
                  pred_ref, topic_ref, bsm_ref, h_ref, hsnew_ref,
                  idx_s, topic_v, bsm_v, alpha_v, hnew_v, acc_v):
    k = pl.program_id(0)

    @pl.when(k == 0)
    def _init():
        topic = ex_e_ref[...] @ WrT_ref[...] + br_ref[...]
        topic_ref[...] = topic
        topic_v[...] = topic
        beta = topic @ vsT_ref[...]

        iota_t = jax.lax.broadcasted_iota(jnp.int32, (1, _T), 1)
        iota_k = jax.lax.broadcasted_iota(jnp.int32, (1, _K), 1)

        def body(i, carry):
            b, vals = carry
            m = jnp.max(b)
            im = jnp.min(jnp.where(b == m, iota_t, _T))
            b = jnp.where(iota_t == im, _NEG, b)
            vals = jnp.where(iota_k == i, m, vals)
            idx_s[i] = im
            return b, vals

        vals0 = jnp.full((1, _K), _NEG, jnp.float32)
        _, vals = jax.lax.fori_loop(0, _K, body, (beta, vals0))
        e = jnp.exp(vals - jnp.max(vals))
        bsm = e / jnp.sum(e)
        bsm_ref[...] = bsm
        bsm_v[...] = bsm

        kn = co_e_ref[...] @ WkT_ref[...] + bk_ref[...]
        al = kn @ kmT_ref[...]
        ea = jnp.exp(al - jnp.max(al))
        alpha = ea / jnp.sum(ea)
        alpha_v[...] = alpha

        g_row = (topic @ WihTv_ref[...]
                 + score_ref[0, 0] * WihTs_ref[...])
        alpha_col = alpha.reshape(_KL, 1)
        gi = alpha_col * g_row + bih_ref[...]
        hprev = h0_ref[...]
        gh = hprev @ WhhT_ref[...] + bhh_ref[...]
        r = jax.nn.sigmoid(gi[:, :_H] + gh[:, :_H])
        z = jax.nn.sigmoid(gi[:, _H:2 * _H] + gh[:, _H:2 * _H])
        n = jnp.tanh(gi[:, 2 * _H:] + r * gh[:, 2 * _H:])
        hnew = (1.0 - z) * n + z * hprev
        hnew_v[...] = hnew
        h_ref[...] = hnew.reshape(1, _KL, _H)
        acc_v[...] = jnp.zeros((_KL, _H), jnp.float32)

    @pl.when(k < _NB)
    def _copy_and_accum():
        hsnew_ref[...] = hs_ref[...]
        iota_k = jax.lax.broadcasted_iota(jnp.int32, (1, _K), 1)

        def scan(i, carry):
            t = idx_s[i]
            j = t - k * _BT

            @pl.when((j >= 0) & (j < _BT))
            def _hit():
                w = jnp.sum(jnp.where(iota_k == i, bsm_v[...], 0.0))
                acc_v[...] = acc_v[...] + w * hs_ref[j]
            return carry

        jax.lax.fori_loop(0, _K, scan, 0)

    @pl.when(k == _NB)
    def _final():
        hsnew_ref[0:1] = hnew_v[...].reshape(1, _KL, _H)
        hkp = alpha_v[...] @ acc_v[...]
        pred_ref[...] = (
            jnp.sum(topic_v[...] * Wsv_ref[...], axis=1, keepdims=True)
            + jnp.sum(hkp * Wsh_ref[...], axis=1, keepdims=True)
            + bs_ref[...])


def kernel(co_e, ex_e, score, time, h0, vs, hs, W_resize, b_resize, Wk, bk,
           know_mem, Ws, bs, W_ih, W_hh, b_ih, b_hh):
    co_e2 = co_e.reshape(1, _KL)
    score2 = score.reshape(1, 1)
    h02 = h0.reshape(_KL, _H)
    vsT = vs.T
    WrT = W_resize.T
    br2 = b_resize.reshape(1, _TS)
    WkT = Wk.T
    bk2 = bk.reshape(1, _KE)
    kmT = know_mem.T
    WihT = W_ih.T
    bih2 = b_ih.reshape(1, 3 * _H)
    WhhT = W_hh.T
    bhh2 = b_hh.reshape(1, 3 * _H)
    Wsv = Ws[:, :_TS]
    Wsh = Ws[:, _TS:]
    bs2 = bs.reshape(1, 1)

    full = lambda *shape: pl.BlockSpec(shape, lambda k: (0,) * len(shape))
    pred, topic, bsm, h, hs_new = pl.pallas_call(
        _fused_kernel,
        grid=(_NB + 1,),
        in_specs=[
            full(1, _EX), full(1, _KL), full(1, 1), full(_KL, _H),
            full(_TS, _T), full(_EX, _TS), full(1, _TS), full(_KL, _KE),
            full(1, _KE), full(_KE, _KL), full(_TS, 3 * _H), full(1, 3 * _H),
            full(1, 3 * _H), full(_H, 3 * _H), full(1, 3 * _H),
            full(1, _TS), full(1, _H), full(1, 1),
            pl.BlockSpec((_BT, _KL, _H),
                         lambda k: (jnp.minimum(k, _NB - 1), 0, 0)),
        ],
        out_specs=[
            pl.BlockSpec((1, 1), lambda k: (0, 0)),
            pl.BlockSpec((1, _TS), lambda k: (0, 0)),
            pl.BlockSpec((1, _K), lambda k: (0, 0)),
            pl.BlockSpec((1, _KL, _H), lambda k: (0, 0, 0)),
            pl.BlockSpec((_BT, _KL, _H), lambda k: (k, 0, 0)),
        ],
        out_shape=[
            jax.ShapeDtypeStruct((1, 1), jnp.float32),
            jax.ShapeDtypeStruct((1, _TS), jnp.float32),
            jax.ShapeDtypeStruct((1, _K), jnp.float32),
            jax.ShapeDtypeStruct((1, _KL, _H), jnp.float32),
            jax.ShapeDtypeStruct((_T + 1, _KL, _H), jnp.float32),
        ],
        scratch_shapes=[
            pltpu.SMEM((_K,), jnp.int32),
            pltpu.VMEM((1, _TS), jnp.float32),
            pltpu.VMEM((1, _K), jnp.float32),
            pltpu.VMEM((1, _KL), jnp.float32),
            pltpu.VMEM((_KL, _H), jnp.float32),
            pltpu.VMEM((_KL, _H), jnp.float32),
        ],
    )(ex_e, co_e2, score2, h02, vsT, WrT, br2, WkT, bk2, kmT,
      WihT[:_TS], WihT[_TS:], bih2, WhhT, bhh2, Wsv, Wsh, bs2, hs)

    vs_new = jnp.concatenate([vs, topic], axis=0)
    return (pred.reshape(1), h, vs_new, hs_new, bsm)
